# baseline (speedup 1.0000x reference)
import functools

import jax
import jax.numpy as jnp
from jax import lax
from jax.experimental import pallas as pl
from jax.experimental.pallas import tpu as pltpu
from jax.experimental.pallas import tpu_sc as plsc

_BT = 128
_ET = 8
_LANES = 16


@functools.lru_cache(maxsize=None)
def _build(batch, hist, n_items, n_layers, num_emb, emb_dim, nc, ns):
  nw = nc * ns
  b_per_w = batch // nw
  nblk = b_per_w // _BT
  nch = nblk * hist
  assert batch == b_per_w * nw and b_per_w == nblk * _BT
  assert nch % 2 == 0 and nch >= 4
  assert emb_dim % _ET == 0 and emb_dim % (2 * _LANES) == 0

  mesh = plsc.VectorSubcoreMesh(
      core_axis_name="c", subcore_axis_name="s", num_cores=nc,
      num_subcores=ns)

  @functools.partial(
      pl.kernel,
      out_type=jax.ShapeDtypeStruct(
          (hist, emb_dim // _ET, batch // _BT, _ET, _BT), jnp.float32),
      mesh=mesh,
      compiler_params=pltpu.CompilerParams(
          use_tc_tiling_on_sc=False, needs_layout_passes=False),
      scratch_types=[
          pltpu.VMEM((2, _BT), jnp.int32),
          pltpu.VMEM((2, n_layers, _BT), jnp.int32),
          pltpu.VMEM((2, n_layers, _BT), jnp.int32),
          pltpu.VMEM((n_layers, num_emb, emb_dim), jnp.bfloat16),
          pltpu.VMEM((2, emb_dim // _ET, _ET, _BT + 1), jnp.float32),
          pltpu.SemaphoreType.DMA,
          pltpu.SemaphoreType.DMA,
          pltpu.SemaphoreType.DMA,
          pltpu.SemaphoreType.DMA,
          pltpu.SemaphoreType.DMA,
          pltpu.SemaphoreType.DMA,
      ],
  )
  def k(ids_hbm, ilids_hbm, *tables_and_rest):
    tabs = tables_and_rest[:n_layers]
    (out_hbm, idsv, fiv, cwv, tabv, outv,
     is0, is1, ls0, ls1, os0, os1) = tables_and_rest[n_layers:]
    isem = (is0, is1)
    lsem = (ls0, ls1)
    osem = (os0, os1)
    wid = lax.axis_index("s") * nc + lax.axis_index("c")

    iota = lax.iota(jnp.int32, _LANES)

    def blk_h(g):
      return g // hist, g % hist

    def ids_off(g):
      blk, h = blk_h(g)
      return pl.multiple_of(h * batch + wid * b_per_w + blk * _BT, _BT)

    def issue_ids(g, p):
      pltpu.async_copy(
          ids_hbm.at[pl.ds(ids_off(g), _BT)], idsv.at[p], isem[p])

    def wait_ids(g, p):
      pltpu.make_async_copy(
          ids_hbm.at[pl.ds(ids_off(g), _BT)], idsv.at[p], isem[p]).wait()

    def fi_and_cw(p):
      for j in range(_BT // _LANES):
        sl = pl.ds(j * _LANES, _LANES)
        idv = idsv[p, sl]
        for l in range(n_layers):
          fiv[p, l, sl] = idv + (l * n_items)
      for l in range(n_layers):
        pltpu.async_copy(
            ilids_hbm.at[fiv.at[p].at[l]], cwv.at[p].at[l], lsem[p])

    def wait_cw(p):
      for l in range(n_layers):
        pltpu.make_async_copy(
            ilids_hbm.at[fiv.at[p].at[l]], cwv.at[p].at[l],
            lsem[p]).wait()

    def do_sum(p):
      nw32 = emb_dim // (2 * _LANES)
      eidx = []
      for cc in range(nw32):
        for off in range(2):
          ev = iota * 2 + (cc * 2 * _LANES + off)
          eidx.append((lax.div(ev, _ET), lax.rem(ev, _ET)))

      @plsc.parallel_loop(0, _BT // _LANES, unroll=2)
      def _(j):
        base = j * _LANES
        bv0 = jnp.full((_LANES,), 0, jnp.int32) + base
        cws = [cwv[p, l, pl.ds(base, _LANES)] for l in range(n_layers)]
        for ti in range(_LANES):
          bv = bv0 + ti
          rows = [cws[l][ti] for l in range(n_layers)]
          for cc in range(nw32):
            sl = pl.ds(cc * 2 * _LANES, 2 * _LANES)
            v = tabv[0, rows[0], sl]
            for l in range(1, n_layers):
              v = v + tabv[l, rows[l], sl]
            va, vb = plsc.unpack(v, format=plsc.PackFormat.INTERLEAVED)
            eta, eia = eidx[cc * 2]
            etb, eib = eidx[cc * 2 + 1]
            plsc.store_scatter(outv.at[p], [eta, eia, bv], va)
            plsc.store_scatter(outv.at[p], [etb, eib, bv], vb)

    def store_out(g, p):
      blk, h = blk_h(g)
      bt = wid * nblk + blk
      pltpu.async_copy(outv.at[p].at[:, :, pl.ds(0, _BT)],
                       out_hbm.at[h, :, bt, :, :], osem[p])

    def wait_out(g, p):
      blk, h = blk_h(g)
      bt = wid * nblk + blk
      pltpu.make_async_copy(
          outv.at[p].at[:, :, pl.ds(0, _BT)],
          out_hbm.at[h, :, bt, :, :], osem[p]).wait()

    for l in range(n_layers):
      pltpu.sync_copy(tabs[l], tabv.at[l])

    issue_ids(0, 0)
    issue_ids(1, 1)
    wait_ids(0, 0)
    fi_and_cw(0)
    wait_ids(1, 1)
    fi_and_cw(1)
    issue_ids(2, 0)

    def step(g, p):
      q = 1 - p

      @pl.when(g + 3 < nch)
      def _():
        issue_ids(g + 3, q)

      wait_cw(p)

      @pl.when(g >= 2)
      def _():
        wait_out(g - 2, p)

      do_sum(p)
      store_out(g, p)

      @pl.when(g + 2 < nch)
      def _():
        wait_ids(g + 2, p)
        fi_and_cw(p)

    @pl.loop(0, nch, step=2)
    def _(g):
      step(g, 0)
      step(g + 1, 1)

    wait_out(nch - 2, 0)
    wait_out(nch - 1, 1)

  return k


def kernel(ids, item_layer_ids, emb_table):
  batch, hist = ids.shape
  n_items, n_layers = item_layer_ids.shape
  num_emb = emb_table.shape[0] // n_layers
  emb_dim = emb_table.shape[1]
  info = plsc.get_sparse_core_info()
  fn = _build(batch, hist, n_items, n_layers, num_emb, emb_dim,
              info.num_cores, info.num_subcores)
  ids_cm = ids.astype(jnp.int32).T.reshape(-1)
  ilids_cm = item_layer_ids.astype(jnp.int32).T.reshape(-1)
  emb = emb_table.astype(jnp.bfloat16)
  tabs = [emb[l * num_emb:(l + 1) * num_emb] for l in range(n_layers)]
  out5d = fn(ids_cm, ilids_cm, *tabs)
  out = jnp.transpose(out5d, (2, 4, 0, 1, 3)).reshape(batch, hist, emb_dim)
  return out


# speedup vs baseline: 1.9645x; 1.9645x over previous
import functools

import jax
import jax.numpy as jnp
from jax import lax
from jax.experimental import pallas as pl
from jax.experimental.pallas import tpu as pltpu
from jax.experimental.pallas import tpu_sc as plsc

_BT = 128
_ET = 8
_LANES = 16


@functools.lru_cache(maxsize=None)
def _build(batch, hist, n_items, n_layers, num_emb, emb_dim, nc, ns):
  nw = nc * ns
  b_per_w = batch // nw
  nblk = b_per_w // _BT
  nch = nblk * hist
  assert batch == b_per_w * nw and b_per_w == nblk * _BT
  assert nch % 2 == 0 and nch >= 4
  assert emb_dim % _ET == 0 and emb_dim % (2 * _LANES) == 0

  mesh = plsc.VectorSubcoreMesh(
      core_axis_name="c", subcore_axis_name="s", num_cores=nc,
      num_subcores=ns)

  @functools.partial(
      pl.kernel,
      out_type=jax.ShapeDtypeStruct(
          (hist, emb_dim // _ET, batch // _BT, _ET, _BT), jnp.float32),
      mesh=mesh,
      compiler_params=pltpu.CompilerParams(
          use_tc_tiling_on_sc=False, needs_layout_passes=False),
      scratch_types=[
          pltpu.VMEM((2, _BT), jnp.int32),
          pltpu.VMEM((2, n_layers, _BT), jnp.int32),
          pltpu.VMEM((2, n_layers, _BT), jnp.int32),
          pltpu.VMEM((2, n_layers, _BT, emb_dim), jnp.bfloat16),
          pltpu.VMEM_SHARED((n_layers, num_emb, emb_dim), jnp.bfloat16),
          pltpu.VMEM((2, emb_dim // _ET, _ET, _BT + 1), jnp.float32),
          pltpu.SemaphoreType.DMA,
          pltpu.SemaphoreType.DMA,
          pltpu.SemaphoreType.DMA,
          pltpu.SemaphoreType.DMA,
          pltpu.SemaphoreType.DMA,
          pltpu.SemaphoreType.DMA,
          pltpu.SemaphoreType.DMA,
          pltpu.SemaphoreType.DMA,
      ],
  )
  def k(ids_hbm, ilids_hbm, *tables_and_rest):
    tabs = tables_and_rest[:n_layers]
    (out_hbm, idsv, fiv, cwv, rowsv, sptab, outv,
     is0, is1, ls0, ls1, rs0, rs1, os0, os1) = tables_and_rest[n_layers:]
    isem = (is0, is1)
    lsem = (ls0, ls1)
    rsem = (rs0, rs1)
    osem = (os0, os1)
    wid = lax.axis_index("s") * nc + lax.axis_index("c")

    iota = lax.iota(jnp.int32, _LANES)

    def blk_h(g):
      return g // hist, g % hist

    def ids_off(g):
      blk, h = blk_h(g)
      return pl.multiple_of(h * batch + wid * b_per_w + blk * _BT, _BT)

    def issue_ids(g, p):
      pltpu.async_copy(
          ids_hbm.at[pl.ds(ids_off(g), _BT)], idsv.at[p], isem[p])

    def wait_ids(g, p):
      pltpu.make_async_copy(
          ids_hbm.at[pl.ds(ids_off(g), _BT)], idsv.at[p], isem[p]).wait()

    def fi_and_cw(p):
      for j in range(_BT // _LANES):
        sl = pl.ds(j * _LANES, _LANES)
        idv = idsv[p, sl]
        for l in range(n_layers):
          fiv[p, l, sl] = idv + (l * n_items)
      for l in range(n_layers):
        pltpu.async_copy(
            ilids_hbm.at[fiv.at[p].at[l]], cwv.at[p].at[l], lsem[p])

    def wait_cw(p):
      for l in range(n_layers):
        pltpu.make_async_copy(
            ilids_hbm.at[fiv.at[p].at[l]], cwv.at[p].at[l],
            lsem[p]).wait()

    def issue_rows(p):
      for l in range(n_layers):
        pltpu.async_copy(
            sptab.at[l].at[cwv.at[p].at[l]], rowsv.at[p].at[l], rsem[p])

    def wait_rows(p):
      for l in range(n_layers):
        pltpu.make_async_copy(
            sptab.at[l].at[cwv.at[p].at[l]], rowsv.at[p].at[l],
            rsem[p]).wait()

    def do_sum(p):
      nw32 = emb_dim // (2 * _LANES)
      eidx = []
      for cc in range(nw32):
        for off in range(2):
          ev = iota * 2 + (cc * 2 * _LANES + off)
          eidx.append((lax.div(ev, _ET), lax.rem(ev, _ET)))

      @plsc.parallel_loop(0, _BT, unroll=8)
      def _(t):
        bv = jnp.full((_LANES,), 0, jnp.int32) + t
        for cc in range(nw32):
          sl = pl.ds(cc * 2 * _LANES, 2 * _LANES)
          v = rowsv[p, 0, t, sl]
          for l in range(1, n_layers):
            v = v + rowsv[p, l, t, sl]
          va, vb = plsc.unpack(v, format=plsc.PackFormat.INTERLEAVED)
          eta, eia = eidx[cc * 2]
          etb, eib = eidx[cc * 2 + 1]
          plsc.store_scatter(outv.at[p], [eta, eia, bv], va)
          plsc.store_scatter(outv.at[p], [etb, eib, bv], vb)

    def store_out(g, p):
      blk, h = blk_h(g)
      bt = wid * nblk + blk
      pltpu.async_copy(outv.at[p].at[:, :, pl.ds(0, _BT)],
                       out_hbm.at[h, :, bt, :, :], osem[p])

    def wait_out(g, p):
      blk, h = blk_h(g)
      bt = wid * nblk + blk
      pltpu.make_async_copy(
          outv.at[p].at[:, :, pl.ds(0, _BT)],
          out_hbm.at[h, :, bt, :, :], osem[p]).wait()

    @pl.when(lax.axis_index("s") == 0)
    def _():
      for l in range(n_layers):
        pltpu.sync_copy(tabs[l], sptab.at[l])

    plsc.subcore_barrier()

    issue_ids(0, 0)
    issue_ids(1, 1)
    wait_ids(0, 0)
    fi_and_cw(0)
    wait_ids(1, 1)
    fi_and_cw(1)
    issue_ids(2, 0)
    wait_cw(0)
    issue_rows(0)

    def step(g, p):
      q = 1 - p

      @pl.when(g + 3 < nch)
      def _():
        issue_ids(g + 3, q)

      @pl.when(g + 1 < nch)
      def _():
        wait_cw(q)
        issue_rows(q)

      wait_rows(p)

      @pl.when(g + 2 < nch)
      def _():
        wait_ids(g + 2, p)
        fi_and_cw(p)

      @pl.when(g >= 2)
      def _():
        wait_out(g - 2, p)

      do_sum(p)
      store_out(g, p)

    @pl.loop(0, nch, step=2)
    def _(g):
      step(g, 0)
      step(g + 1, 1)

    wait_out(nch - 2, 0)
    wait_out(nch - 1, 1)

  return k


def kernel(ids, item_layer_ids, emb_table):
  batch, hist = ids.shape
  n_items, n_layers = item_layer_ids.shape
  num_emb = emb_table.shape[0] // n_layers
  emb_dim = emb_table.shape[1]
  info = plsc.get_sparse_core_info()
  fn = _build(batch, hist, n_items, n_layers, num_emb, emb_dim,
              info.num_cores, info.num_subcores)
  ids_cm = ids.astype(jnp.int32).T.reshape(-1)
  ilids_cm = item_layer_ids.astype(jnp.int32).T.reshape(-1)
  emb = emb_table.astype(jnp.bfloat16)
  tabs = [emb[l * num_emb:(l + 1) * num_emb] for l in range(n_layers)]
  out5d = fn(ids_cm, ilids_cm, *tabs)
  out = jnp.transpose(out5d, (2, 4, 0, 1, 3)).reshape(batch, hist, emb_dim)
  return out
